# baseline (speedup 1.0000x reference)
import functools

import jax
import jax.numpy as jnp
from jax import lax
from jax.experimental import pallas as pl
from jax.experimental.pallas import tpu as pltpu
from jax.experimental.pallas import tpu_sc as plsc

D = 64
B = 4096
S = 200

NC = 2
NS = 16
NW = NC * NS
BPW = B // NW
GB = 2
ROWS = GB * S
ITERS = BPW // GB
CHUNK = 80
NCH = ROWS // CHUNK
IDS_PW = BPW * S
GRP = 16 // GB


@functools.partial(
    pl.kernel,
    out_type=jax.ShapeDtypeStruct((B,), jnp.float32),
    mesh=plsc.VectorSubcoreMesh(core_axis_name="c", subcore_axis_name="s"),
    compiler_params=pltpu.CompilerParams(
        needs_layout_passes=False, use_tc_tiling_on_sc=False),
    scratch_types=[
        pltpu.VMEM((IDS_PW,), jnp.int32),
        pltpu.VMEM((ROWS, D), jnp.float32),
        pltpu.VMEM((ROWS, D), jnp.float32),
        pltpu.VMEM((D,), jnp.float32),
        pltpu.VMEM((16,), jnp.float32),
        pltpu.VMEM((BPW,), jnp.float32),
        pltpu.VMEM((256,), jnp.float32),
        pltpu.SemaphoreType.DMA,
        pltpu.SemaphoreType.DMA,
    ],
)
def _sc_mlp(ids_hbm, table_hbm, w_hbm, b_hbm, out_hbm,
            ids_v, rows0, rows1, w_v, b_v, out_v, tbuf, sem0, sem1):
    cid = lax.axis_index("c")
    sid = lax.axis_index("s")
    wid = sid * NC + cid

    pltpu.sync_copy(ids_hbm.at[pl.ds(wid * IDS_PW, IDS_PW)], ids_v)
    pltpu.sync_copy(w_hbm, w_v)
    pltpu.sync_copy(b_hbm, b_v)

    def dbl(i, carry):
        ids_v[pl.ds(pl.multiple_of(i * 16, 16), 16)] = (
            ids_v[pl.ds(pl.multiple_of(i * 16, 16), 16)] * 2)
        return carry

    lax.fori_loop(0, IDS_PW // 16, dbl, jnp.int32(0))

    inv = jnp.float32(1.0 / S)
    ws = [w_v[pl.ds(16 * j, 16)] * inv for j in range(4)]
    bvec = b_v[...]
    lanes = lax.iota(jnp.int32, 16)
    base16 = lanes * 16
    zero = jnp.zeros((16,), jnp.float32)

    def start_gather(it, buf, sem):
        off = it * ROWS
        for k in range(NCH):
            src = ids_v.at[pl.ds(pl.multiple_of(off + k * CHUNK, CHUNK), CHUNK)]
            pltpu.async_copy(table_hbm.at[src], buf.at[pl.ds(k * CHUNK, CHUNK)], sem)

    def wait_gather(buf, sem):
        pltpu.make_async_copy(table_hbm.at[pl.ds(0, ROWS)], buf, sem).wait()

    def compute(buf, it):
        for e in range(GB):
            base = e * S

            def srow(i, accs, _base=base, _buf=buf):
                a0, a1, a2, a3 = accs
                r = _base + i * 4
                for u in range(4):
                    a0 = a0 + _buf[r + u, pl.ds(0, 16)]
                    a1 = a1 + _buf[r + u, pl.ds(16, 16)]
                    a2 = a2 + _buf[r + u, pl.ds(32, 16)]
                    a3 = a3 + _buf[r + u, pl.ds(48, 16)]
                return (a0, a1, a2, a3)

            a0, a1, a2, a3 = lax.fori_loop(0, S // 4, srow,
                                           (zero, zero, zero, zero))
            t = a0 * ws[0] + a1 * ws[1] + a2 * ws[2] + a3 * ws[3]
            lane = GB * (it % GRP) + e
            tbuf[pl.ds(pl.multiple_of(lane * 16, 16), 16)] = t

        grp = it // GRP

        @pl.when(it % GRP == GRP - 1)
        def _():
            acc = bvec
            for j in range(16):
                acc = acc + plsc.load_gather(tbuf, [base16 + j])
            y = jnp.float32(1.0) / (jnp.float32(1.0) + jnp.exp(-acc))
            out_v[pl.ds(pl.multiple_of(grp * 16, 16), 16)] = y

    start_gather(0, rows0, sem0)

    def super_body(g, carry):
        it0 = 2 * g
        start_gather(it0 + 1, rows1, sem1)
        wait_gather(rows0, sem0)
        compute(rows0, it0)

        @pl.when(g < ITERS // 2 - 1)
        def _():
            start_gather(it0 + 2, rows0, sem0)

        wait_gather(rows1, sem1)
        compute(rows1, it0 + 1)
        return carry

    lax.fori_loop(0, ITERS // 2, super_body, jnp.int32(0))

    pltpu.sync_copy(out_v, out_hbm.at[pl.ds(wid * BPW, BPW)])


TCK = 2048


@functools.partial(
    pl.pallas_call,
    grid=((1000000 + TCK - 1) // TCK,),
    in_specs=[pl.BlockSpec((D, TCK), lambda i: (0, i))],
    out_specs=pl.BlockSpec((TCK, 128), lambda i: (i, 0)),
    out_shape=jax.ShapeDtypeStruct((1000000, 128), jnp.float32),
)
def _tc_detile(tt_ref, out_ref):
    eye = jnp.eye(D, dtype=jnp.float32)
    out_ref[:, :D] = jax.lax.dot_general(
        tt_ref[...], eye, (((0,), (0,)), ((), ())),
        preferred_element_type=jnp.float32)


def kernel(ids, table, W, b):
    ids_flat = ids.reshape(-1).astype(jnp.int32)
    table2 = _tc_detile(table.T).reshape(-1, 64)
    w_flat = W.reshape(-1).astype(jnp.float32)
    b16 = jnp.broadcast_to(b.astype(jnp.float32).reshape(-1), (16,))
    out = _sc_mlp(ids_flat, table2, w_flat, b16)
    return out.reshape(B, 1)


# speedup vs baseline: 1.5611x; 1.5611x over previous
import functools

import jax
import jax.numpy as jnp
from jax import lax
from jax.experimental import pallas as pl
from jax.experimental.pallas import tpu as pltpu
from jax.experimental.pallas import tpu_sc as plsc

D = 64
B = 4096
S = 200

NC = 2
NS = 16
NW = NC * NS
BPW = B // NW
GB = 2
ROWS = GB * S
ITERS = BPW // GB
CHUNK = 80
NCH = ROWS // CHUNK
IDS_PW = BPW * S
GRP = 16 // GB


@functools.partial(
    pl.kernel,
    out_type=jax.ShapeDtypeStruct((B,), jnp.float32),
    mesh=plsc.VectorSubcoreMesh(core_axis_name="c", subcore_axis_name="s"),
    compiler_params=pltpu.CompilerParams(
        needs_layout_passes=False, use_tc_tiling_on_sc=False),
    scratch_types=[
        pltpu.VMEM((IDS_PW,), jnp.int32),
        pltpu.VMEM((ROWS, D), jnp.float32),
        pltpu.VMEM((ROWS, D), jnp.float32),
        pltpu.VMEM((D,), jnp.float32),
        pltpu.VMEM((16,), jnp.float32),
        pltpu.VMEM((BPW,), jnp.float32),
        pltpu.VMEM((256,), jnp.float32),
        pltpu.SemaphoreType.DMA,
        pltpu.SemaphoreType.DMA,
    ],
)
def _sc_mlp(ids_hbm, table_hbm, w_hbm, b_hbm, out_hbm,
            ids_v, rows0, rows1, w_v, b_v, out_v, tbuf, sem0, sem1):
    cid = lax.axis_index("c")
    sid = lax.axis_index("s")
    wid = sid * NC + cid

    pltpu.sync_copy(ids_hbm.at[pl.ds(wid * IDS_PW, IDS_PW)], ids_v)
    pltpu.sync_copy(w_hbm, w_v)
    pltpu.sync_copy(b_hbm, b_v)

    def dbl(i, carry):
        ids_v[pl.ds(pl.multiple_of(i * 16, 16), 16)] = (
            ids_v[pl.ds(pl.multiple_of(i * 16, 16), 16)] * 2)
        return carry

    lax.fori_loop(0, IDS_PW // 16, dbl, jnp.int32(0))

    inv = jnp.float32(1.0 / S)
    ws = [w_v[pl.ds(16 * j, 16)] * inv for j in range(4)]
    bvec = b_v[...]
    lanes = lax.iota(jnp.int32, 16)
    base16 = lanes * 16
    zero = jnp.zeros((16,), jnp.float32)

    def start_gather(it, buf, sem):
        off = it * ROWS
        for k in range(NCH):
            src = ids_v.at[pl.ds(pl.multiple_of(off + k * CHUNK, CHUNK), CHUNK)]
            pltpu.async_copy(table_hbm.at[src], buf.at[pl.ds(k * CHUNK, CHUNK)], sem)

    def wait_gather(buf, sem):
        pltpu.make_async_copy(table_hbm.at[pl.ds(0, ROWS)], buf, sem).wait()

    def compute(buf, it):
        for e in range(GB):
            base = e * S

            def srow(i, accs, _base=base, _buf=buf):
                a0, a1, a2, a3 = accs
                r = _base + i * 4
                for u in range(4):
                    a0 = a0 + _buf[r + u, pl.ds(0, 16)]
                    a1 = a1 + _buf[r + u, pl.ds(16, 16)]
                    a2 = a2 + _buf[r + u, pl.ds(32, 16)]
                    a3 = a3 + _buf[r + u, pl.ds(48, 16)]
                return (a0, a1, a2, a3)

            a0, a1, a2, a3 = lax.fori_loop(0, S // 4, srow,
                                           (zero, zero, zero, zero))
            t = a0 * ws[0] + a1 * ws[1] + a2 * ws[2] + a3 * ws[3]
            lane = GB * (it % GRP) + e
            tbuf[pl.ds(pl.multiple_of(lane * 16, 16), 16)] = t

        grp = it // GRP

        @pl.when(it % GRP == GRP - 1)
        def _():
            acc = bvec
            for j in range(16):
                acc = acc + plsc.load_gather(tbuf, [base16 + j])
            y = jnp.float32(1.0) / (jnp.float32(1.0) + jnp.exp(-acc))
            out_v[pl.ds(pl.multiple_of(grp * 16, 16), 16)] = y

    start_gather(0, rows0, sem0)

    def super_body(g, carry):
        it0 = 2 * g
        start_gather(it0 + 1, rows1, sem1)
        wait_gather(rows0, sem0)
        compute(rows0, it0)

        @pl.when(g < ITERS // 2 - 1)
        def _():
            start_gather(it0 + 2, rows0, sem0)

        wait_gather(rows1, sem1)
        compute(rows1, it0 + 1)
        return carry

    lax.fori_loop(0, ITERS // 2, super_body, jnp.int32(0))

    pltpu.sync_copy(out_v, out_hbm.at[pl.ds(wid * BPW, BPW)])


TCK = 8192


@functools.partial(
    pl.pallas_call,
    grid=((1000000 + TCK - 1) // TCK,),
    in_specs=[pl.BlockSpec((D, TCK), lambda i: (0, i))],
    out_specs=pl.BlockSpec((TCK, 128), lambda i: (i, 0)),
    out_shape=jax.ShapeDtypeStruct((1000000, 128), jnp.float32),
)
def _tc_detile(tt_ref, out_ref):
    out_ref[:, :D] = tt_ref[...].T


def kernel(ids, table, W, b):
    ids_flat = ids.reshape(-1).astype(jnp.int32)
    table2 = _tc_detile(table.T).reshape(-1, 64)
    w_flat = W.reshape(-1).astype(jnp.float32)
    b16 = jnp.broadcast_to(b.astype(jnp.float32).reshape(-1), (16,))
    out = _sc_mlp(ids_flat, table2, w_flat, b16)
    return out.reshape(B, 1)


# speedup vs baseline: 1.6401x; 1.0506x over previous
import functools

import jax
import jax.numpy as jnp
from jax import lax
from jax.experimental import pallas as pl
from jax.experimental.pallas import tpu as pltpu
from jax.experimental.pallas import tpu_sc as plsc

D = 64
B = 4096
S = 200

NC = 2
NS = 16
NW = NC * NS
BPW = B // NW
GB = 2
ROWS = GB * S
ITERS = BPW // GB
CHUNK = 80
NCH = ROWS // CHUNK
IDS_PW = BPW * S
GRP = 16 // GB


@functools.partial(
    pl.kernel,
    out_type=jax.ShapeDtypeStruct((B,), jnp.float32),
    mesh=plsc.VectorSubcoreMesh(core_axis_name="c", subcore_axis_name="s"),
    compiler_params=pltpu.CompilerParams(
        needs_layout_passes=False, use_tc_tiling_on_sc=False),
    scratch_types=[
        pltpu.VMEM((IDS_PW,), jnp.int32),
        pltpu.VMEM((ROWS, D), jnp.float32),
        pltpu.VMEM((ROWS, D), jnp.float32),
        pltpu.VMEM((D,), jnp.float32),
        pltpu.VMEM((16,), jnp.float32),
        pltpu.VMEM((BPW,), jnp.float32),
        pltpu.VMEM((256,), jnp.float32),
        pltpu.SemaphoreType.DMA,
        pltpu.SemaphoreType.DMA,
    ],
)
def _sc_mlp(ids_hbm, table_hbm, w_hbm, b_hbm, out_hbm,
            ids_v, rows0, rows1, w_v, b_v, out_v, tbuf, sem0, sem1):
    cid = lax.axis_index("c")
    sid = lax.axis_index("s")
    wid = sid * NC + cid

    pltpu.sync_copy(ids_hbm.at[pl.ds(wid * IDS_PW, IDS_PW)], ids_v)
    pltpu.sync_copy(w_hbm, w_v)
    pltpu.sync_copy(b_hbm, b_v)

    def dbl(i, carry):
        ids_v[pl.ds(pl.multiple_of(i * 16, 16), 16)] = (
            ids_v[pl.ds(pl.multiple_of(i * 16, 16), 16)] * 2)
        return carry

    lax.fori_loop(0, IDS_PW // 16, dbl, jnp.int32(0))

    inv = jnp.float32(1.0 / S)
    ws = [w_v[pl.ds(16 * j, 16)] * inv for j in range(4)]
    bvec = b_v[...]
    lanes = lax.iota(jnp.int32, 16)
    base16 = lanes * 16
    zero = jnp.zeros((16,), jnp.float32)

    def start_gather(it, buf, sem):
        off = it * ROWS
        for k in range(NCH):
            src = ids_v.at[pl.ds(pl.multiple_of(off + k * CHUNK, CHUNK), CHUNK)]
            pltpu.async_copy(table_hbm.at[src], buf.at[pl.ds(k * CHUNK, CHUNK)], sem)

    def wait_gather(buf, sem):
        pltpu.make_async_copy(table_hbm.at[pl.ds(0, ROWS)], buf, sem).wait()

    def compute(buf, it):
        for e in range(GB):
            base = e * S

            def srow(i, accs, _base=base, _buf=buf):
                a0, a1, a2, a3 = accs
                r = _base + i * 4
                for u in range(4):
                    a0 = a0 + _buf[r + u, pl.ds(0, 16)]
                    a1 = a1 + _buf[r + u, pl.ds(16, 16)]
                    a2 = a2 + _buf[r + u, pl.ds(32, 16)]
                    a3 = a3 + _buf[r + u, pl.ds(48, 16)]
                return (a0, a1, a2, a3)

            a0, a1, a2, a3 = lax.fori_loop(0, S // 4, srow,
                                           (zero, zero, zero, zero))
            t = a0 * ws[0] + a1 * ws[1] + a2 * ws[2] + a3 * ws[3]
            lane = GB * (it % GRP) + e
            tbuf[pl.ds(pl.multiple_of(lane * 16, 16), 16)] = t

        grp = it // GRP

        @pl.when(it % GRP == GRP - 1)
        def _():
            acc = bvec
            for j in range(16):
                acc = acc + plsc.load_gather(tbuf, [base16 + j])
            y = jnp.float32(1.0) / (jnp.float32(1.0) + jnp.exp(-acc))
            out_v[pl.ds(pl.multiple_of(grp * 16, 16), 16)] = y

    start_gather(0, rows0, sem0)

    def super_body(g, carry):
        it0 = 2 * g
        start_gather(it0 + 1, rows1, sem1)
        wait_gather(rows0, sem0)
        compute(rows0, it0)

        @pl.when(g < ITERS // 2 - 1)
        def _():
            start_gather(it0 + 2, rows0, sem0)

        wait_gather(rows1, sem1)
        compute(rows1, it0 + 1)
        return carry

    lax.fori_loop(0, ITERS // 2, super_body, jnp.int32(0))

    pltpu.sync_copy(out_v, out_hbm.at[pl.ds(wid * BPW, BPW)])


TCK = 16384


@functools.partial(
    pl.pallas_call,
    grid=((1000000 + TCK - 1) // TCK,),
    in_specs=[pl.BlockSpec((D, TCK), lambda i: (0, i))],
    out_specs=pl.BlockSpec((TCK, 128), lambda i: (i, 0)),
    out_shape=jax.ShapeDtypeStruct((1000000, 128), jnp.float32),
)
def _tc_detile(tt_ref, out_ref):
    out_ref[:, :D] = tt_ref[...].T


def kernel(ids, table, W, b):
    ids_flat = ids.reshape(-1).astype(jnp.int32)
    table2 = _tc_detile(table.T).reshape(-1, 64)
    w_flat = W.reshape(-1).astype(jnp.float32)
    b16 = jnp.broadcast_to(b.astype(jnp.float32).reshape(-1), (16,))
    out = _sc_mlp(ids_flat, table2, w_flat, b16)
    return out.reshape(B, 1)


# speedup vs baseline: 1.6723x; 1.0196x over previous
import functools

import jax
import jax.numpy as jnp
from jax import lax
from jax.experimental import pallas as pl
from jax.experimental.pallas import tpu as pltpu
from jax.experimental.pallas import tpu_sc as plsc

D = 64
B = 4096
S = 200

NC = 2
NS = 16
NW = NC * NS
BPW = B // NW
GB = 2
ROWS = GB * S
ITERS = BPW // GB
CHUNK = 80
NCH = ROWS // CHUNK
IDS_PW = BPW * S
GRP = 16 // GB


@functools.partial(
    pl.kernel,
    out_type=jax.ShapeDtypeStruct((B,), jnp.float32),
    mesh=plsc.VectorSubcoreMesh(core_axis_name="c", subcore_axis_name="s"),
    compiler_params=pltpu.CompilerParams(
        needs_layout_passes=False, use_tc_tiling_on_sc=False),
    scratch_types=[
        pltpu.VMEM((IDS_PW,), jnp.int32),
        pltpu.VMEM((ROWS, D), jnp.float32),
        pltpu.VMEM((ROWS, D), jnp.float32),
        pltpu.VMEM((D,), jnp.float32),
        pltpu.VMEM((16,), jnp.float32),
        pltpu.VMEM((BPW,), jnp.float32),
        pltpu.VMEM((256,), jnp.float32),
        pltpu.SemaphoreType.DMA,
        pltpu.SemaphoreType.DMA,
    ],
)
def _sc_mlp(ids_hbm, table_hbm, w_hbm, b_hbm, out_hbm,
            ids_v, rows0, rows1, w_v, b_v, out_v, tbuf, sem0, sem1):
    cid = lax.axis_index("c")
    sid = lax.axis_index("s")
    wid = sid * NC + cid

    pltpu.sync_copy(ids_hbm.at[pl.ds(wid * IDS_PW, IDS_PW)], ids_v)
    pltpu.sync_copy(w_hbm, w_v)
    pltpu.sync_copy(b_hbm, b_v)

    def dbl(i, carry):
        ids_v[pl.ds(pl.multiple_of(i * 16, 16), 16)] = (
            ids_v[pl.ds(pl.multiple_of(i * 16, 16), 16)] * 2)
        return carry

    lax.fori_loop(0, IDS_PW // 16, dbl, jnp.int32(0))

    inv = jnp.float32(1.0 / S)
    ws = [w_v[pl.ds(16 * j, 16)] * inv for j in range(4)]
    bvec = b_v[...]
    lanes = lax.iota(jnp.int32, 16)
    base16 = lanes * 16
    zero = jnp.zeros((16,), jnp.float32)

    def start_gather(it, buf, sem):
        off = it * ROWS
        for k in range(NCH):
            src = ids_v.at[pl.ds(pl.multiple_of(off + k * CHUNK, CHUNK), CHUNK)]
            pltpu.async_copy(table_hbm.at[src], buf.at[pl.ds(k * CHUNK, CHUNK)], sem)

    def wait_gather(buf, sem):
        pltpu.make_async_copy(table_hbm.at[pl.ds(0, ROWS)], buf, sem).wait()

    def compute(buf, it):
        for e in range(GB):
            base = e * S

            def srow(i, accs, _base=base, _buf=buf):
                a0, a1, a2, a3 = accs
                r = _base + i * 4
                for u in range(4):
                    a0 = a0 + _buf[r + u, pl.ds(0, 16)]
                    a1 = a1 + _buf[r + u, pl.ds(16, 16)]
                    a2 = a2 + _buf[r + u, pl.ds(32, 16)]
                    a3 = a3 + _buf[r + u, pl.ds(48, 16)]
                return (a0, a1, a2, a3)

            a0, a1, a2, a3 = lax.fori_loop(0, S // 4, srow,
                                           (zero, zero, zero, zero))
            t = a0 * ws[0] + a1 * ws[1] + a2 * ws[2] + a3 * ws[3]
            lane = GB * (it % GRP) + e
            tbuf[pl.ds(pl.multiple_of(lane * 16, 16), 16)] = t

        grp = it // GRP

        @pl.when(it % GRP == GRP - 1)
        def _():
            acc = bvec
            for j in range(16):
                acc = acc + plsc.load_gather(tbuf, [base16 + j])
            y = jnp.float32(1.0) / (jnp.float32(1.0) + jnp.exp(-acc))
            out_v[pl.ds(pl.multiple_of(grp * 16, 16), 16)] = y

    start_gather(0, rows0, sem0)

    def super_body(g, carry):
        it0 = 2 * g
        start_gather(it0 + 1, rows1, sem1)
        wait_gather(rows0, sem0)
        compute(rows0, it0)

        @pl.when(g < ITERS // 2 - 1)
        def _():
            start_gather(it0 + 2, rows0, sem0)

        wait_gather(rows1, sem1)
        compute(rows1, it0 + 1)
        return carry

    lax.fori_loop(0, ITERS // 2, super_body, jnp.int32(0))

    pltpu.sync_copy(out_v, out_hbm.at[pl.ds(wid * BPW, BPW)])


TCK = 32768


@functools.partial(
    pl.pallas_call,
    grid=((1000000 + TCK - 1) // TCK,),
    in_specs=[pl.BlockSpec((D, TCK), lambda i: (0, i))],
    out_specs=pl.BlockSpec((TCK, 128), lambda i: (i, 0)),
    out_shape=jax.ShapeDtypeStruct((1000000, 128), jnp.float32),
)
def _tc_detile(tt_ref, out_ref):
    out_ref[:, :D] = tt_ref[...].T


def kernel(ids, table, W, b):
    ids_flat = ids.reshape(-1).astype(jnp.int32)
    table2 = _tc_detile(table.T).reshape(-1, 64)
    w_flat = W.reshape(-1).astype(jnp.float32)
    b16 = jnp.broadcast_to(b.astype(jnp.float32).reshape(-1), (16,))
    out = _sc_mlp(ids_flat, table2, w_flat, b16)
    return out.reshape(B, 1)


# speedup vs baseline: 1.6744x; 1.0013x over previous
import functools

import jax
import jax.numpy as jnp
from jax import lax
from jax.experimental import pallas as pl
from jax.experimental.pallas import tpu as pltpu
from jax.experimental.pallas import tpu_sc as plsc

D = 64
B = 4096
S = 200

NC = 2
NS = 16
NW = NC * NS
BPW = B // NW
GB = 2
ROWS = GB * S
ITERS = BPW // GB
CHUNKS = (128, 128, 128, 16)
IDS_PW = BPW * S
GRP = 16 // GB


@functools.partial(
    pl.kernel,
    out_type=jax.ShapeDtypeStruct((B,), jnp.float32),
    mesh=plsc.VectorSubcoreMesh(core_axis_name="c", subcore_axis_name="s"),
    compiler_params=pltpu.CompilerParams(
        needs_layout_passes=False, use_tc_tiling_on_sc=False),
    scratch_types=[
        pltpu.VMEM((IDS_PW,), jnp.int32),
        pltpu.VMEM((ROWS, D), jnp.float32),
        pltpu.VMEM((ROWS, D), jnp.float32),
        pltpu.VMEM((D,), jnp.float32),
        pltpu.VMEM((16,), jnp.float32),
        pltpu.VMEM((BPW,), jnp.float32),
        pltpu.VMEM((256,), jnp.float32),
        pltpu.SemaphoreType.DMA,
        pltpu.SemaphoreType.DMA,
    ],
)
def _sc_mlp(ids_hbm, table_hbm, w_hbm, b_hbm, out_hbm,
            ids_v, rows0, rows1, w_v, b_v, out_v, tbuf, sem0, sem1):
    cid = lax.axis_index("c")
    sid = lax.axis_index("s")
    wid = sid * NC + cid

    pltpu.sync_copy(ids_hbm.at[pl.ds(wid * IDS_PW, IDS_PW)], ids_v)
    pltpu.sync_copy(w_hbm, w_v)
    pltpu.sync_copy(b_hbm, b_v)

    def dbl(i, carry):
        ids_v[pl.ds(pl.multiple_of(i * 16, 16), 16)] = (
            ids_v[pl.ds(pl.multiple_of(i * 16, 16), 16)] * 2)
        return carry

    lax.fori_loop(0, IDS_PW // 16, dbl, jnp.int32(0))

    inv = jnp.float32(1.0 / S)
    ws = [w_v[pl.ds(16 * j, 16)] * inv for j in range(4)]
    bvec = b_v[...]
    lanes = lax.iota(jnp.int32, 16)
    base16 = lanes * 16
    zero = jnp.zeros((16,), jnp.float32)

    def start_gather(it, buf, sem):
        off = it * ROWS
        pos = 0
        for ck in CHUNKS:
            src = ids_v.at[pl.ds(pl.multiple_of(off + pos, 8), ck)]
            pltpu.async_copy(table_hbm.at[src], buf.at[pl.ds(pos, ck)], sem)
            pos += ck

    def wait_gather(buf, sem):
        pltpu.make_async_copy(table_hbm.at[pl.ds(0, ROWS)], buf, sem).wait()

    def compute(buf, it):
        for e in range(GB):
            base = e * S

            def srow(i, accs, _base=base, _buf=buf):
                a0, a1, a2, a3 = accs
                r = _base + i * 4
                for u in range(4):
                    a0 = a0 + _buf[r + u, pl.ds(0, 16)]
                    a1 = a1 + _buf[r + u, pl.ds(16, 16)]
                    a2 = a2 + _buf[r + u, pl.ds(32, 16)]
                    a3 = a3 + _buf[r + u, pl.ds(48, 16)]
                return (a0, a1, a2, a3)

            a0, a1, a2, a3 = lax.fori_loop(0, S // 4, srow,
                                           (zero, zero, zero, zero))
            t = a0 * ws[0] + a1 * ws[1] + a2 * ws[2] + a3 * ws[3]
            lane = GB * (it % GRP) + e
            tbuf[pl.ds(pl.multiple_of(lane * 16, 16), 16)] = t

        grp = it // GRP

        @pl.when(it % GRP == GRP - 1)
        def _():
            acc = bvec
            for j in range(16):
                acc = acc + plsc.load_gather(tbuf, [base16 + j])
            y = jnp.float32(1.0) / (jnp.float32(1.0) + jnp.exp(-acc))
            out_v[pl.ds(pl.multiple_of(grp * 16, 16), 16)] = y

    start_gather(0, rows0, sem0)

    def super_body(g, carry):
        it0 = 2 * g
        start_gather(it0 + 1, rows1, sem1)
        wait_gather(rows0, sem0)
        compute(rows0, it0)

        @pl.when(g < ITERS // 2 - 1)
        def _():
            start_gather(it0 + 2, rows0, sem0)

        wait_gather(rows1, sem1)
        compute(rows1, it0 + 1)
        return carry

    lax.fori_loop(0, ITERS // 2, super_body, jnp.int32(0))

    pltpu.sync_copy(out_v, out_hbm.at[pl.ds(wid * BPW, BPW)])


TCK = 32768


@functools.partial(
    pl.pallas_call,
    grid=((1000000 + TCK - 1) // TCK,),
    in_specs=[pl.BlockSpec((D, TCK), lambda i: (0, i))],
    out_specs=pl.BlockSpec((TCK, 128), lambda i: (i, 0)),
    out_shape=jax.ShapeDtypeStruct((1000000, 128), jnp.float32),
)
def _tc_detile(tt_ref, out_ref):
    out_ref[:, :D] = tt_ref[...].T


def kernel(ids, table, W, b):
    ids_flat = ids.reshape(-1).astype(jnp.int32)
    table2 = _tc_detile(table.T).reshape(-1, 64)
    w_flat = W.reshape(-1).astype(jnp.float32)
    b16 = jnp.broadcast_to(b.astype(jnp.float32).reshape(-1), (16,))
    out = _sc_mlp(ids_flat, table2, w_flat, b16)
    return out.reshape(B, 1)
